# baseline (speedup 1.0000x reference)
import jax
import jax.numpy as jnp
from jax.experimental import pallas as pl

N = 4096
D_IN = 768
D_DICT = 16384
TOP_K = 64

TN = 1024
TD = 2048


def _mm_body(x_ref, w_ref, b_ref, p_ref, bm_ref):
    acc = jax.lax.dot_general(
        x_ref[...], w_ref[...], (((1,), (0,)), ((), ())),
        preferred_element_type=jnp.float32,
        precision=jax.lax.Precision.DEFAULT)
    p = jnp.maximum(acc + b_ref[...], 0.0)
    p_ref[...] = p
    bm_ref[...] = jnp.max(p.reshape(TN // 128, 128, TD), axis=1)


def _encode(x, W_enc, b_enc):
    grid = (N // TN, D_DICT // TD)
    return pl.pallas_call(
        _mm_body,
        grid=grid,
        in_specs=[
            pl.BlockSpec((TN, D_IN), lambda i, j: (i, 0)),
            pl.BlockSpec((D_IN, TD), lambda i, j: (0, j)),
            pl.BlockSpec((1, TD), lambda i, j: (0, j)),
        ],
        out_specs=[
            pl.BlockSpec((TN, TD), lambda i, j: (i, j)),
            pl.BlockSpec((TN // 128, TD), lambda i, j: (i, j)),
        ],
        out_shape=[
            jax.ShapeDtypeStruct((N, D_DICT), jnp.float32),
            jax.ShapeDtypeStruct((N // 128, D_DICT), jnp.float32),
        ],
    )(x, W_enc, b_enc.reshape(1, D_DICT))


def kernel(layer_idx, x, W_enc, b_enc):
    P, BM = _encode(x, W_enc, b_enc)
    del BM
    flat = P.reshape(-1)
    k = TOP_K * N
    vals, idx = jax.lax.top_k(flat, k)
    out = jnp.zeros_like(flat).at[idx].set(vals)
    return out.reshape(P.shape)


# speedup vs baseline: 68.6019x; 68.6019x over previous
import functools

import jax
import jax.numpy as jnp
from jax import lax
from jax.experimental import pallas as pl
from jax.experimental.pallas import tpu as pltpu
from jax.experimental.pallas import tpu_sc as plsc

N = 4096
D_IN = 768
D_DICT = 16384
TOP_K = 64
K_TOTAL = TOP_K * N
TOTAL = N * D_DICT

TN, TD = 1024, 2048
RB = 128

_NC, _NS = 2, 16
_NW = _NC * _NS
_WROWS = N // _NW
_CROWS = 2
_NCH = _WROWS // _CROWS
_SLAB = 32768


def _mm_body(x_ref, w_ref, b_ref, p_ref, bm_ref):
    acc = lax.dot_general(
        x_ref[...], w_ref[...], (((1,), (0,)), ((), ())),
        preferred_element_type=jnp.float32,
        precision=lax.Precision.DEFAULT)
    p = jnp.maximum(acc + b_ref[...], 0.0)
    p_ref[...] = p
    bm_ref[...] = jnp.max(p.reshape(TN // RB, RB, TD), axis=1)


def _encode(x, W_enc, b_enc):
    return pl.pallas_call(
        _mm_body,
        grid=(N // TN, D_DICT // TD),
        in_specs=[
            pl.BlockSpec((TN, D_IN), lambda i, j: (i, 0)),
            pl.BlockSpec((D_IN, TD), lambda i, j: (0, j)),
            pl.BlockSpec((1, TD), lambda i, j: (0, j)),
        ],
        out_specs=[
            pl.BlockSpec((TN, TD), lambda i, j: (i, j)),
            pl.BlockSpec((TN // RB, TD), lambda i, j: (i, j)),
        ],
        out_shape=[
            jax.ShapeDtypeStruct((N, D_DICT), jnp.float32),
            jax.ShapeDtypeStruct((N // RB, D_DICT), jnp.float32),
        ],
    )(x, W_enc, b_enc.reshape(1, D_DICT))


def _make_select(nbits):
    def body(x_ref, t_ref):
        bits = lax.bitcast_convert_type(x_ref[...], jnp.int32)
        phi = jnp.int32(0)
        for b in range(30, 30 - nbits, -1):
            cand = phi | jnp.int32(1 << b)
            cnt = jnp.sum((bits >= cand).astype(jnp.int32))
            phi = jnp.where(cnt >= K_TOTAL, cand, phi)
        t = lax.bitcast_convert_type(phi, jnp.float32)
        t_ref[...] = jnp.full((8, 128), t, jnp.float32)

    def run(x):
        return pl.pallas_call(
            body,
            out_shape=jax.ShapeDtypeStruct((8, 128), jnp.float32),
        )(x)
    return run


_select_lo = _make_select(19)
_select_hi = _make_select(31)


_SLAB_L = _SLAB // 16


def _compact_body(p_hbm, tlo_hbm, vals_hbm, cnt_hbm, tlo_v, buf, slab, cnt_v):
    wid = lax.axis_index("s") * _NC + lax.axis_index("c")
    row0 = wid * _WROWS
    pltpu.sync_copy(tlo_hbm, tlo_v)
    tlo = tlo_v[...]
    lane_base = lax.iota(jnp.int32, 16) * _SLAB_L

    def zbody(i, carry):
        slab[pl.ds(i * 16, 16)] = jnp.zeros((16,), jnp.float32)
        return carry
    lax.fori_loop(0, _SLAB // 16, zbody, jnp.int32(0))

    zeros16 = jnp.zeros((16,), jnp.int32)
    ones16 = jnp.ones((16,), jnp.int32)

    def chunk(c, cnt_vec):
        pltpu.sync_copy(p_hbm.at[pl.ds(row0 + c * _CROWS, _CROWS)], buf)

        def scan(i, cnt_vec):
            def one(r, cnt_vec):
                v = buf[r, pl.ds(i * 16, 16)]
                m = jnp.logical_and(v >= tlo, cnt_vec < _SLAB_L)
                plsc.store_scatter(slab, [lane_base + cnt_vec], v, mask=m)
                return cnt_vec + jnp.where(m, ones16, zeros16)
            for r in range(_CROWS):
                cnt_vec = one(r, cnt_vec)
            return cnt_vec
        return lax.fori_loop(0, D_DICT // 16, scan, cnt_vec)

    cnt_vec = lax.fori_loop(0, _NCH, chunk, zeros16)
    pltpu.sync_copy(slab, vals_hbm.at[wid])
    cnt_v[...] = cnt_vec
    pltpu.sync_copy(cnt_v, cnt_hbm.at[wid])


_compact = pl.kernel(
    _compact_body,
    out_type=[
        jax.ShapeDtypeStruct((_NW, _SLAB), jnp.float32),
        jax.ShapeDtypeStruct((_NW, 16), jnp.int32),
    ],
    mesh=plsc.VectorSubcoreMesh(core_axis_name="c", subcore_axis_name="s"),
    compiler_params=pltpu.CompilerParams(needs_layout_passes=False),
    scratch_types=[
        pltpu.VMEM((16,), jnp.float32),
        pltpu.VMEM((_CROWS, D_DICT), jnp.float32),
        pltpu.VMEM((_SLAB,), jnp.float32),
        pltpu.VMEM((16,), jnp.int32),
    ],
)


def _mask_body(t_ref, p_ref, o_ref):
    t = t_ref[0, 0]
    p = p_ref[...]
    o_ref[...] = jnp.where(p >= t, p, 0.0)


def _mask(t11, P):
    BN, BD = 512, 4096
    return pl.pallas_call(
        _mask_body,
        grid=(N // BN, D_DICT // BD),
        in_specs=[
            pl.BlockSpec(memory_space=pltpu.MemorySpace.SMEM),
            pl.BlockSpec((BN, BD), lambda i, j: (i, j)),
        ],
        out_specs=pl.BlockSpec((BN, BD), lambda i, j: (i, j)),
        out_shape=jax.ShapeDtypeStruct((N, D_DICT), jnp.float32),
    )(t11, P)


def kernel(layer_idx, x, W_enc, b_enc):
    P, BM = _encode(x, W_enc, b_enc)
    tlo_tile = _select_lo(BM)
    vals, _cnts = _compact(P, tlo_tile[0, :16])
    t_tile = _select_hi(vals)
    return _mask(t_tile[:1, :1], P)


# speedup vs baseline: 79.3000x; 1.1559x over previous
import functools

import jax
import jax.numpy as jnp
from jax import lax
from jax.experimental import pallas as pl
from jax.experimental.pallas import tpu as pltpu
from jax.experimental.pallas import tpu_sc as plsc

N = 4096
D_IN = 768
D_DICT = 16384
TOP_K = 64
K_TOTAL = TOP_K * N
TOTAL = N * D_DICT

TN, TD = 1024, 2048
RB = 128

_NC, _NS = 2, 16
_NW = _NC * _NS
_WROWS = N // _NW
_CROWS = 2
_NCH = _WROWS // _CROWS
_SLAB = 32768


def _mm_body(x_ref, w_ref, b_ref, p_ref, bm_ref):
    acc = lax.dot_general(
        x_ref[...], w_ref[...], (((1,), (0,)), ((), ())),
        preferred_element_type=jnp.float32,
        precision=lax.Precision.DEFAULT)
    p = jnp.maximum(acc + b_ref[...], 0.0)
    p_ref[...] = p
    bm_ref[...] = jnp.max(p.reshape(TN // RB, RB, TD), axis=1)


def _encode(x, W_enc, b_enc):
    return pl.pallas_call(
        _mm_body,
        grid=(N // TN, D_DICT // TD),
        in_specs=[
            pl.BlockSpec((TN, D_IN), lambda i, j: (i, 0)),
            pl.BlockSpec((D_IN, TD), lambda i, j: (0, j)),
            pl.BlockSpec((1, TD), lambda i, j: (0, j)),
        ],
        out_specs=[
            pl.BlockSpec((TN, TD), lambda i, j: (i, j)),
            pl.BlockSpec((TN // RB, TD), lambda i, j: (i, j)),
        ],
        out_shape=[
            jax.ShapeDtypeStruct((N, D_DICT), jnp.float32),
            jax.ShapeDtypeStruct((N // RB, D_DICT), jnp.float32),
        ],
    )(x, W_enc, b_enc.reshape(1, D_DICT))


def _make_select(nbits):
    def body(x_ref, t_ref):
        bits = lax.bitcast_convert_type(x_ref[...], jnp.int32)
        phi = jnp.int32(0)
        for b in range(30, 30 - nbits, -1):
            cand = phi | jnp.int32(1 << b)
            cnt = jnp.sum((bits >= cand).astype(jnp.int32))
            phi = jnp.where(cnt >= K_TOTAL, cand, phi)
        t = lax.bitcast_convert_type(phi, jnp.float32)
        t_ref[...] = jnp.full((8, 128), t, jnp.float32)

    def run(x):
        return pl.pallas_call(
            body,
            out_shape=jax.ShapeDtypeStruct((8, 128), jnp.float32),
        )(x)
    return run


_select_lo = _make_select(19)
_select_hi = _make_select(31)


_SLAB_L = _SLAB // 16


_UNROLL = 8


def _compact_body(p_hbm, tlo_hbm, vals_hbm, cnt_hbm, tlo_v, buf, slab, cnt_v,
                  sem0, sem1):
    wid = lax.axis_index("s") * _NC + lax.axis_index("c")
    row0 = wid * _WROWS
    pltpu.sync_copy(tlo_hbm, tlo_v)
    tlo = tlo_v[...]
    lane_base = lax.iota(jnp.int32, 16) * _SLAB_L
    sems = (sem0, sem1)

    def zbody(i, carry):
        for u in range(_UNROLL):
            slab[pl.ds((i * _UNROLL + u) * 16, 16)] = jnp.zeros(
                (16,), jnp.float32)
        return carry
    lax.fori_loop(0, _SLAB // 16 // _UNROLL, zbody, jnp.int32(0))

    def chunk_src(c):
        return p_hbm.at[pl.ds(row0 + c * _CROWS, _CROWS)]

    pltpu.async_copy(chunk_src(0), buf.at[0], sem0)
    pltpu.async_copy(chunk_src(1), buf.at[1], sem1)

    def outer(c2, cnt_vec):
        for b in range(2):
            c = c2 * 2 + b
            pltpu.make_async_copy(chunk_src(c), buf.at[b], sems[b]).wait()
            for r in range(_CROWS):
                def scan(i, cnt_vec, r=r, b=b):
                    for u in range(_UNROLL):
                        v = buf[b, r, pl.ds((i * _UNROLL + u) * 16, 16)]
                        m = jnp.logical_and(v >= tlo, cnt_vec < _SLAB_L)
                        plsc.store_scatter(
                            slab, [lane_base + cnt_vec], v, mask=m)
                        cnt_vec = cnt_vec + m.astype(jnp.int32)
                    return cnt_vec
                cnt_vec = lax.fori_loop(
                    0, D_DICT // 16 // _UNROLL, scan, cnt_vec)

            @pl.when(c + 2 < _NCH)
            def _(c=c, b=b):
                pltpu.async_copy(chunk_src(c + 2), buf.at[b], sems[b])
        return cnt_vec

    cnt_vec = lax.fori_loop(0, _NCH // 2, outer, jnp.zeros((16,), jnp.int32))
    pltpu.sync_copy(slab, vals_hbm.at[wid])
    cnt_v[...] = cnt_vec
    pltpu.sync_copy(cnt_v, cnt_hbm.at[wid])


_compact = pl.kernel(
    _compact_body,
    out_type=[
        jax.ShapeDtypeStruct((_NW, _SLAB), jnp.float32),
        jax.ShapeDtypeStruct((_NW, 16), jnp.int32),
    ],
    mesh=plsc.VectorSubcoreMesh(core_axis_name="c", subcore_axis_name="s"),
    compiler_params=pltpu.CompilerParams(needs_layout_passes=False),
    scratch_types=[
        pltpu.VMEM((16,), jnp.float32),
        pltpu.VMEM((2, _CROWS, D_DICT), jnp.float32),
        pltpu.VMEM((_SLAB,), jnp.float32),
        pltpu.VMEM((16,), jnp.int32),
        pltpu.SemaphoreType.DMA,
        pltpu.SemaphoreType.DMA,
    ],
)


def _mask_body(t_ref, p_ref, o_ref):
    t = t_ref[0, 0]
    p = p_ref[...]
    o_ref[...] = jnp.where(p >= t, p, 0.0)


def _mask(t11, P):
    BN, BD = 512, 4096
    return pl.pallas_call(
        _mask_body,
        grid=(N // BN, D_DICT // BD),
        in_specs=[
            pl.BlockSpec(memory_space=pltpu.MemorySpace.SMEM),
            pl.BlockSpec((BN, BD), lambda i, j: (i, j)),
        ],
        out_specs=pl.BlockSpec((BN, BD), lambda i, j: (i, j)),
        out_shape=jax.ShapeDtypeStruct((N, D_DICT), jnp.float32),
    )(t11, P)


def kernel(layer_idx, x, W_enc, b_enc):
    P, BM = _encode(x, W_enc, b_enc)
    tlo_tile = _select_lo(BM)
    vals, _cnts = _compact(P, tlo_tile[0, :16])
    t_tile = _select_hi(vals)
    return _mask(t_tile[:1, :1], P)


# speedup vs baseline: 84.0685x; 1.0601x over previous
import functools

import jax
import jax.numpy as jnp
from jax import lax
from jax.experimental import pallas as pl
from jax.experimental.pallas import tpu as pltpu
from jax.experimental.pallas import tpu_sc as plsc

N = 4096
D_IN = 768
D_DICT = 16384
TOP_K = 64
K_TOTAL = TOP_K * N
TOTAL = N * D_DICT

TN, TD = 1024, 2048
RB = 128

_NC, _NS = 2, 16
_NW = _NC * _NS
_WROWS = N // _NW
_CROWS = 2
_NCH = _WROWS // _CROWS
_SLAB = 32768


def _mm_body(x_ref, w_ref, b_ref, p_ref, bm_ref):
    acc = lax.dot_general(
        x_ref[...], w_ref[...], (((1,), (0,)), ((), ())),
        preferred_element_type=jnp.float32,
        precision=lax.Precision.DEFAULT)
    p = jnp.maximum(acc + b_ref[...], 0.0)
    p_ref[...] = p
    bm_ref[...] = jnp.max(p.reshape(TN // RB, RB, TD), axis=1)


def _encode(x, W_enc, b_enc):
    return pl.pallas_call(
        _mm_body,
        grid=(N // TN, D_DICT // TD),
        in_specs=[
            pl.BlockSpec((TN, D_IN), lambda i, j: (i, 0)),
            pl.BlockSpec((D_IN, TD), lambda i, j: (0, j)),
            pl.BlockSpec((1, TD), lambda i, j: (0, j)),
        ],
        out_specs=[
            pl.BlockSpec((TN, TD), lambda i, j: (i, j)),
            pl.BlockSpec((TN // RB, TD), lambda i, j: (i, j)),
        ],
        out_shape=[
            jax.ShapeDtypeStruct((N, D_DICT), jnp.float32),
            jax.ShapeDtypeStruct((N // RB, D_DICT), jnp.float32),
        ],
    )(x, W_enc, b_enc.reshape(1, D_DICT))


def _make_select(nbits):
    def body(x_ref, t_ref):
        bits = lax.bitcast_convert_type(x_ref[...], jnp.int32)
        phi = jnp.int32(0)
        for b in range(30, 30 - nbits, -1):
            cand = phi | jnp.int32(1 << b)
            cnt = jnp.sum((bits >= cand).astype(jnp.int32))
            phi = jnp.where(cnt >= K_TOTAL, cand, phi)
        t = lax.bitcast_convert_type(phi, jnp.float32)
        t_ref[...] = jnp.full((8, 128), t, jnp.float32)

    def run(x):
        return pl.pallas_call(
            body,
            out_shape=jax.ShapeDtypeStruct((8, 128), jnp.float32),
        )(x)
    return run


_select_lo = _make_select(19)
_select_hi = _make_select(31)


_UNROLL = 8
_SUB = _SLAB // 16 // _UNROLL


def _compact_body(p_hbm, tlo_hbm, vals_hbm, cnt_hbm, tlo_v, buf, slab, cnt_v,
                  sem0, sem1):
    wid = lax.axis_index("s") * _NC + lax.axis_index("c")
    row0 = wid * _WROWS
    pltpu.sync_copy(tlo_hbm, tlo_v)
    tlo = tlo_v[...]
    iota16 = lax.iota(jnp.int32, 16)
    bases = tuple(iota16 * _SUB + (u * 16 * _SUB) for u in range(_UNROLL))
    limits = tuple(b + (_SUB - 1) for b in bases)
    sems = (sem0, sem1)

    def zbody(i, carry):
        for u in range(_UNROLL):
            slab[pl.ds((i * _UNROLL + u) * 16, 16)] = jnp.zeros(
                (16,), jnp.float32)
        return carry
    lax.fori_loop(0, _SLAB // 16 // _UNROLL, zbody, jnp.int32(0))

    def chunk_src(c):
        return p_hbm.at[pl.ds(row0 + c * _CROWS, _CROWS)]

    pltpu.async_copy(chunk_src(0), buf.at[0], sem0)
    pltpu.async_copy(chunk_src(1), buf.at[1], sem1)

    def outer(c2, idxs):
        for b in range(2):
            c = c2 * 2 + b
            pltpu.make_async_copy(chunk_src(c), buf.at[b], sems[b]).wait()
            for r in range(_CROWS):
                def scan(i, idxs, r=r, b=b):
                    new = []
                    for u in range(_UNROLL):
                        v = buf[b, r, pl.ds((i * _UNROLL + u) * 16, 16)]
                        m = v >= tlo
                        ic = jnp.minimum(idxs[u], limits[u])
                        plsc.store_scatter(slab, [ic], v, mask=m)
                        new.append(idxs[u] + m.astype(jnp.int32))
                    return tuple(new)
                idxs = lax.fori_loop(
                    0, D_DICT // 16 // _UNROLL, scan, idxs)

            @pl.when(c + 2 < _NCH)
            def _(c=c, b=b):
                pltpu.async_copy(chunk_src(c + 2), buf.at[b], sems[b])
        return idxs

    idxs = lax.fori_loop(0, _NCH // 2, outer, bases)
    pltpu.sync_copy(slab, vals_hbm.at[wid])
    total = bases[0] - bases[0]
    for u in range(_UNROLL):
        total = total + (idxs[u] - bases[u])
    cnt_v[...] = total
    pltpu.sync_copy(cnt_v, cnt_hbm.at[wid])


_compact = pl.kernel(
    _compact_body,
    out_type=[
        jax.ShapeDtypeStruct((_NW, _SLAB), jnp.float32),
        jax.ShapeDtypeStruct((_NW, 16), jnp.int32),
    ],
    mesh=plsc.VectorSubcoreMesh(core_axis_name="c", subcore_axis_name="s"),
    compiler_params=pltpu.CompilerParams(needs_layout_passes=False),
    scratch_types=[
        pltpu.VMEM((16,), jnp.float32),
        pltpu.VMEM((2, _CROWS, D_DICT), jnp.float32),
        pltpu.VMEM((_SLAB,), jnp.float32),
        pltpu.VMEM((16,), jnp.int32),
        pltpu.SemaphoreType.DMA,
        pltpu.SemaphoreType.DMA,
    ],
)


def _mask_body(t_ref, p_ref, o_ref):
    t = t_ref[0, 0]
    p = p_ref[...]
    o_ref[...] = jnp.where(p >= t, p, 0.0)


def _mask(t11, P):
    BN, BD = 512, 4096
    return pl.pallas_call(
        _mask_body,
        grid=(N // BN, D_DICT // BD),
        in_specs=[
            pl.BlockSpec(memory_space=pltpu.MemorySpace.SMEM),
            pl.BlockSpec((BN, BD), lambda i, j: (i, j)),
        ],
        out_specs=pl.BlockSpec((BN, BD), lambda i, j: (i, j)),
        out_shape=jax.ShapeDtypeStruct((N, D_DICT), jnp.float32),
    )(t11, P)


def kernel(layer_idx, x, W_enc, b_enc):
    P, BM = _encode(x, W_enc, b_enc)
    tlo_tile = _select_lo(BM)
    vals, _cnts = _compact(P, tlo_tile[0, :16])
    t_tile = _select_hi(vals)
    return _mask(t_tile[:1, :1], P)
